# baseline (device time: 107781 ns/iter reference)
import jax
import jax.numpy as jnp
from jax import lax
from jax.experimental import pallas as pl
from jax.experimental.pallas import tpu as pltpu


def kernel(x, dy):
    k, m = x.shape
    _, f = dy.shape
    fh = f // 2
    mh = m // 2

    my_y = lax.axis_index("y")
    dyh = lax.dynamic_slice(dy, (0, my_y * fh), (k, fh))
    p = lax.dot_general(
        x, dyh, (((0,), (0,)), ((), ())),
        preferred_element_type=jnp.float32,
    )

    def body(p_ref, out_ref, rx_ref, sx, rx_sem, sy, ry):
        my_x = lax.axis_index("x")
        my_y = lax.axis_index("y")

        barrier = pltpu.get_barrier_semaphore()
        pl.semaphore_signal(
            barrier, inc=1, device_id=(1 - my_x, my_y),
            device_id_type=pl.DeviceIdType.MESH,
        )
        pl.semaphore_signal(
            barrier, inc=1, device_id=(my_x, 1 - my_y),
            device_id_type=pl.DeviceIdType.MESH,
        )
        pl.semaphore_wait(barrier, 2)

        rdma_x = pltpu.make_async_remote_copy(
            src_ref=p_ref.at[pl.ds((1 - my_x) * mh, mh), :],
            dst_ref=rx_ref,
            send_sem=sx,
            recv_sem=rx_sem,
            device_id=(1 - my_x, my_y),
            device_id_type=pl.DeviceIdType.MESH,
        )
        rdma_x.start()
        rdma_x.wait()

        out_ref[:, pl.ds(my_y * fh, fh)] = (
            p_ref[pl.ds(my_x * mh, mh), :] + rx_ref[:, :]
        )

        rdma_y = pltpu.make_async_remote_copy(
            src_ref=out_ref.at[:, pl.ds(my_y * fh, fh)],
            dst_ref=out_ref.at[:, pl.ds(my_y * fh, fh)],
            send_sem=sy,
            recv_sem=ry,
            device_id=(my_x, 1 - my_y),
            device_id_type=pl.DeviceIdType.MESH,
        )
        rdma_y.start()
        rdma_y.wait()

    return pl.pallas_call(
        body,
        out_shape=jax.ShapeDtypeStruct((mh, f), jnp.float32),
        in_specs=[pl.BlockSpec(memory_space=pltpu.VMEM)],
        out_specs=pl.BlockSpec(memory_space=pltpu.VMEM),
        scratch_shapes=[
            pltpu.VMEM((mh, fh), jnp.float32),
            pltpu.SemaphoreType.DMA,
            pltpu.SemaphoreType.DMA,
            pltpu.SemaphoreType.DMA,
            pltpu.SemaphoreType.DMA,
        ],
        compiler_params=pltpu.CompilerParams(collective_id=0),
    )(p)


# device time: 68362 ns/iter; 1.5766x vs baseline; 1.5766x over previous
import jax
import jax.numpy as jnp
from jax import lax
from jax.experimental import pallas as pl
from jax.experimental.pallas import tpu as pltpu


def kernel(x, dy):
    k, m = x.shape
    _, f = dy.shape
    fh = f // 2
    mh = m // 2

    my_y = lax.axis_index("y")
    dyh = lax.dynamic_slice(dy, (0, my_y * fh), (k, fh))
    p = lax.dot_general(
        x, dyh, (((0,), (0,)), ((), ())),
        preferred_element_type=jnp.float32,
    )

    NC = 8
    fc = fh // NC

    def body(p_ref, out_ref, rx_ref, sxs, rxs, sys_, rys):
        my_x = lax.axis_index("x")
        my_y = lax.axis_index("y")

        barrier = pltpu.get_barrier_semaphore()
        pl.semaphore_signal(
            barrier, inc=1, device_id=(1 - my_x, my_y),
            device_id_type=pl.DeviceIdType.MESH,
        )
        pl.semaphore_signal(
            barrier, inc=1, device_id=(my_x, 1 - my_y),
            device_id_type=pl.DeviceIdType.MESH,
        )
        pl.semaphore_wait(barrier, 2)

        rdmas_x = []
        for c in range(NC):
            r = pltpu.make_async_remote_copy(
                src_ref=p_ref.at[pl.ds((1 - my_x) * mh, mh), pl.ds(c * fc, fc)],
                dst_ref=rx_ref.at[:, pl.ds(c * fc, fc)],
                send_sem=sxs.at[c],
                recv_sem=rxs.at[c],
                device_id=(1 - my_x, my_y),
                device_id_type=pl.DeviceIdType.MESH,
            )
            r.start()
            rdmas_x.append(r)

        rdmas_y = []
        for c in range(NC):
            rdmas_x[c].wait_recv()
            out_ref[:, pl.ds(my_y * fh + c * fc, fc)] = (
                p_ref[pl.ds(my_x * mh, mh), pl.ds(c * fc, fc)]
                + rx_ref[:, pl.ds(c * fc, fc)]
            )
            r = pltpu.make_async_remote_copy(
                src_ref=out_ref.at[:, pl.ds(my_y * fh + c * fc, fc)],
                dst_ref=out_ref.at[:, pl.ds(my_y * fh + c * fc, fc)],
                send_sem=sys_.at[c],
                recv_sem=rys.at[c],
                device_id=(my_x, 1 - my_y),
                device_id_type=pl.DeviceIdType.MESH,
            )
            r.start()
            rdmas_y.append(r)

        for c in range(NC):
            rdmas_y[c].wait_recv()
            rdmas_y[c].wait_send()
            rdmas_x[c].wait_send()

    return pl.pallas_call(
        body,
        out_shape=jax.ShapeDtypeStruct((mh, f), jnp.float32),
        in_specs=[pl.BlockSpec(memory_space=pltpu.VMEM)],
        out_specs=pl.BlockSpec(memory_space=pltpu.VMEM),
        scratch_shapes=[
            pltpu.VMEM((mh, fh), jnp.float32),
            pltpu.SemaphoreType.DMA((NC,)),
            pltpu.SemaphoreType.DMA((NC,)),
            pltpu.SemaphoreType.DMA((NC,)),
            pltpu.SemaphoreType.DMA((NC,)),
        ],
        compiler_params=pltpu.CompilerParams(collective_id=0),
    )(p)
